# baseline (speedup 1.0000x reference)
import functools

import jax
import jax.numpy as jnp
from jax.experimental import pallas as pl
from jax.experimental.pallas import tpu as pltpu

K = 512
D = 256
HW = 256


def _vq_kernel(z_ref, emb_ref, out_ref):
    z = z_ref[0]

    def body(k, carry):
        best_d, best_i = carry
        e = emb_ref[pl.ds(k, 1), :]
        a = z - e
        sq = a * a
        d = (jnp.sum(sq[:, :128], axis=1, keepdims=True)
             + jnp.sum(sq[:, 128:], axis=1, keepdims=True))
        mask = d < best_d
        best_d = jnp.where(mask, d, best_d)
        best_i = jnp.where(mask, k, best_i)
        return best_d, best_i

    init = (
        jnp.full((HW, 1), jnp.inf, dtype=jnp.float32),
        jnp.zeros((HW, 1), dtype=jnp.int32),
    )
    _, best_i = jax.lax.fori_loop(0, K, body, init)
    out_ref[0] = best_i


@jax.jit
def kernel(z_e_x, emb):
    B = z_e_x.shape[0]
    H, W = z_e_x.shape[2], z_e_x.shape[3]
    zt = z_e_x.reshape(B, D, H * W).transpose(0, 2, 1)
    out = pl.pallas_call(
        _vq_kernel,
        grid=(B,),
        in_specs=[
            pl.BlockSpec((1, H * W, D), lambda b: (b, 0, 0)),
            pl.BlockSpec((K, D), lambda b: (0, 0)),
        ],
        out_specs=pl.BlockSpec((1, H * W, 1), lambda b: (b, 0, 0)),
        out_shape=jax.ShapeDtypeStruct((B, H * W, 1), jnp.int32),
        compiler_params=pltpu.CompilerParams(
            dimension_semantics=("parallel",),
        ),
    )(zt, emb)
    return out.reshape(B, H, W)


# speedup vs baseline: 1.9657x; 1.9657x over previous
import jax
import jax.numpy as jnp
from jax.experimental import pallas as pl
from jax.experimental.pallas import tpu as pltpu

K = 512
D = 256
HW = 256
KB = 8


def _vq_kernel(z_ref, emb_ref, out_ref):
    z = z_ref[0]
    row = jax.lax.broadcasted_iota(jnp.int32, (KB, HW), 0)

    def body(kb, carry):
        best_d, best_i = carry
        base = kb * KB
        eblk = emb_ref[pl.ds(base, KB), :]
        ds = []
        for j in range(KB):
            a = z - eblk[j : j + 1, :]
            sq = a * a
            dj = (jnp.sum(sq[:, :128], axis=1)
                  + jnp.sum(sq[:, 128:], axis=1))
            ds.append(dj.reshape(1, HW))
        d8 = jnp.concatenate(ds, axis=0)
        mask = d8 < best_d
        best_d = jnp.where(mask, d8, best_d)
        best_i = jnp.where(mask, base + row, best_i)
        return best_d, best_i

    init = (
        jnp.full((KB, HW), jnp.inf, dtype=jnp.float32),
        jnp.zeros((KB, HW), dtype=jnp.int32),
    )
    best_d, best_i = jax.lax.fori_loop(0, K // KB, body, init)

    for half in (4, 2, 1):
        d_lo, d_hi = best_d[:half], best_d[half : 2 * half]
        i_lo, i_hi = best_i[:half], best_i[half : 2 * half]
        take_hi = (d_hi < d_lo) | ((d_hi == d_lo) & (i_hi < i_lo))
        best_d = jnp.where(take_hi, d_hi, d_lo)
        best_i = jnp.where(take_hi, i_hi, i_lo)
    out_ref[0] = best_i


@jax.jit
def kernel(z_e_x, emb):
    B = z_e_x.shape[0]
    H, W = z_e_x.shape[2], z_e_x.shape[3]
    zt = z_e_x.reshape(B, D, H * W).transpose(0, 2, 1)
    out = pl.pallas_call(
        _vq_kernel,
        grid=(B,),
        in_specs=[
            pl.BlockSpec((1, H * W, D), lambda b: (b, 0, 0)),
            pl.BlockSpec((K, D), lambda b: (0, 0)),
        ],
        out_specs=pl.BlockSpec((1, 1, H * W), lambda b: (b, 0, 0)),
        out_shape=jax.ShapeDtypeStruct((B, 1, H * W), jnp.int32),
        compiler_params=pltpu.CompilerParams(
            dimension_semantics=("parallel",),
        ),
    )(zt, emb)
    return out.reshape(B, H, W)


# speedup vs baseline: 2.1187x; 1.0779x over previous
import jax
import jax.numpy as jnp
from jax.experimental import pallas as pl
from jax.experimental.pallas import tpu as pltpu

K = 512
D = 256
HW = 256
KB = 16


def _vq_kernel(z_ref, emb_ref, out_ref):
    z = z_ref[0]
    row = jax.lax.broadcasted_iota(jnp.int32, (KB, HW), 0)

    def body(kb, carry):
        best_d, best_i = carry
        base = kb * KB
        eblk = emb_ref[pl.ds(base, KB), :]
        ds = []
        for j in range(KB):
            a = z - eblk[j : j + 1, :]
            sq = a * a
            dj = (jnp.sum(sq[:, :128], axis=1)
                  + jnp.sum(sq[:, 128:], axis=1))
            ds.append(dj.reshape(1, HW))
        d8 = jnp.concatenate(ds, axis=0)
        mask = d8 < best_d
        best_d = jnp.where(mask, d8, best_d)
        best_i = jnp.where(mask, base + row, best_i)
        return best_d, best_i

    init = (
        jnp.full((KB, HW), jnp.inf, dtype=jnp.float32),
        jnp.zeros((KB, HW), dtype=jnp.int32),
    )
    best_d, best_i = jax.lax.fori_loop(0, K // KB, body, init)

    half = KB // 2
    while half >= 1:
        d_lo, d_hi = best_d[:half], best_d[half : 2 * half]
        i_lo, i_hi = best_i[:half], best_i[half : 2 * half]
        take_hi = (d_hi < d_lo) | ((d_hi == d_lo) & (i_hi < i_lo))
        best_d = jnp.where(take_hi, d_hi, d_lo)
        best_i = jnp.where(take_hi, i_hi, i_lo)
        half //= 2
    out_ref[0] = best_i


@jax.jit
def kernel(z_e_x, emb):
    B = z_e_x.shape[0]
    H, W = z_e_x.shape[2], z_e_x.shape[3]
    zt = z_e_x.reshape(B, D, H * W).transpose(0, 2, 1)
    out = pl.pallas_call(
        _vq_kernel,
        grid=(B,),
        in_specs=[
            pl.BlockSpec((1, H * W, D), lambda b: (b, 0, 0)),
            pl.BlockSpec((K, D), lambda b: (0, 0)),
        ],
        out_specs=pl.BlockSpec((1, 1, H * W), lambda b: (b, 0, 0)),
        out_shape=jax.ShapeDtypeStruct((B, 1, H * W), jnp.int32),
        compiler_params=pltpu.CompilerParams(
            dimension_semantics=("parallel",),
        ),
    )(zt, emb)
    return out.reshape(B, H, W)


# speedup vs baseline: 6.9397x; 3.2755x over previous
import jax
import jax.numpy as jnp
from jax.experimental import pallas as pl
from jax.experimental.pallas import tpu as pltpu

K = 512
D = 256
HW = 256
T = 8


def _screen_kernel(zb_ref, emb_ref, ids_ref, s_ref):
    zb = zb_ref[0]
    emb = emb_ref[...]

    esq = emb * emb
    e2 = (jnp.sum(esq[:, :128], axis=1) + jnp.sum(esq[:, 128:], axis=1))

    s = jax.lax.dot_general(
        emb, zb, dimension_numbers=(((1,), (0,)), ((), ())),
        precision=jax.lax.Precision.HIGHEST,
        preferred_element_type=jnp.float32,
    )
    s_ref[...] = e2.reshape(K, 1) - 2.0 * s

    def body(t, carry):
        s = s_ref[...]
        kiota = jax.lax.broadcasted_iota(jnp.int32, (K, HW), 0).astype(
            jnp.float32)
        mv = jnp.min(s, axis=0, keepdims=True)
        idx = jnp.min(jnp.where(s == mv, kiota, jnp.float32(K)),
                      axis=0, keepdims=True)
        ids_ref[0, pl.ds(t, 1), :] = idx.astype(jnp.int32)
        s_ref[...] = jnp.where(kiota == idx, jnp.float32(1e30), s)
        return carry

    jax.lax.fori_loop(0, T, body, 0)


def _refine_kernel(ids_smem_ref, z_ref, emb_ref, ids_vec_ref, out_ref, g_ref):
    z = z_ref[0]

    def gather_row(pos, _):
        for t in range(T):
            idx = ids_smem_ref[0, t, pos]
            g_ref[t, pl.ds(pos, 1), :] = emb_ref[pl.ds(idx, 1), :]
        return 0

    jax.lax.fori_loop(0, HW, gather_row, 0)

    best_d = jnp.full((1, HW), jnp.inf, dtype=jnp.float32)
    best_i = jnp.zeros((1, HW), dtype=jnp.int32)
    for t in range(T):
        a = z - g_ref[t]
        sq = a * a
        d = (jnp.sum(sq[:, :128], axis=1)
             + jnp.sum(sq[:, 128:], axis=1)).reshape(1, HW)
        i = ids_vec_ref[0, t : t + 1, :]
        take = (d < best_d) | ((d == best_d) & (i < best_i))
        best_d = jnp.where(take, d, best_d)
        best_i = jnp.where(take, i, best_i)
    out_ref[0] = best_i


@jax.jit
def kernel(z_e_x, emb):
    B = z_e_x.shape[0]
    H, W = z_e_x.shape[2], z_e_x.shape[3]
    zb = z_e_x.reshape(B, D, H * W)

    ids = pl.pallas_call(
        _screen_kernel,
        grid=(B,),
        in_specs=[
            pl.BlockSpec((1, D, H * W), lambda b: (b, 0, 0)),
            pl.BlockSpec((K, D), lambda b: (0, 0)),
        ],
        out_specs=pl.BlockSpec((1, T, H * W), lambda b: (b, 0, 0)),
        out_shape=jax.ShapeDtypeStruct((B, T, H * W), jnp.int32),
        scratch_shapes=[
            pltpu.VMEM((K, H * W), jnp.float32),
        ],
        compiler_params=pltpu.CompilerParams(
            dimension_semantics=("parallel",),
        ),
    )(zb, emb)

    zt = zb.transpose(0, 2, 1)

    out = pl.pallas_call(
        _refine_kernel,
        grid=(B,),
        in_specs=[
            pl.BlockSpec((1, T, H * W), lambda b: (b, 0, 0),
                         memory_space=pltpu.SMEM),
            pl.BlockSpec((1, H * W, D), lambda b: (b, 0, 0)),
            pl.BlockSpec((K, D), lambda b: (0, 0)),
            pl.BlockSpec((1, T, H * W), lambda b: (b, 0, 0)),
        ],
        out_specs=pl.BlockSpec((1, 1, H * W), lambda b: (b, 0, 0)),
        out_shape=jax.ShapeDtypeStruct((B, 1, H * W), jnp.int32),
        scratch_shapes=[pltpu.VMEM((T, H * W, D), jnp.float32)],
        compiler_params=pltpu.CompilerParams(
            dimension_semantics=("parallel",),
        ),
    )(ids, zt, emb, ids)
    return out.reshape(B, H, W)


# speedup vs baseline: 11.8186x; 1.7030x over previous
import jax
import jax.numpy as jnp
from jax.experimental import pallas as pl
from jax.experimental.pallas import tpu as pltpu

K = 512
D = 256
HW = 256
T = 4


def _screen_kernel(zb_ref, emb_ref, ids_ref, s_ref):
    zb = zb_ref[0]
    emb = emb_ref[...]

    esq = emb * emb
    e2 = (jnp.sum(esq[:, :128], axis=1) + jnp.sum(esq[:, 128:], axis=1))

    s = jax.lax.dot_general(
        emb, zb, dimension_numbers=(((1,), (0,)), ((), ())),
        precision=jax.lax.Precision.HIGHEST,
        preferred_element_type=jnp.float32,
    )
    s_ref[...] = e2.reshape(K, 1) - 2.0 * s

    def body(t, carry):
        s = s_ref[...]
        kiota = jax.lax.broadcasted_iota(jnp.int32, (K, HW), 0).astype(
            jnp.float32)
        mv = jnp.min(s, axis=0, keepdims=True)
        idx = jnp.min(jnp.where(s == mv, kiota, jnp.float32(K)),
                      axis=0, keepdims=True)
        ids_ref[0, pl.ds(t, 1), :] = idx.astype(jnp.int32)
        s_ref[...] = jnp.where(kiota == idx, jnp.float32(1e30), s)
        return carry

    jax.lax.fori_loop(0, T, body, 0)


def _refine_kernel(ids_smem_ref, z_ref, emb_ref, ids_vec_ref, out_ref, g_ref):
    z = z_ref[0]

    def gather_rows(blk, _):
        for u in range(4):
            for t in range(T):
                pos = blk * 4 + u
                idx = ids_smem_ref[0, t, pos]
                g_ref[t, pl.ds(pos, 1), :] = emb_ref[pl.ds(idx, 1), :]
        return 0

    jax.lax.fori_loop(0, HW // 4, gather_rows, 0)

    best_d = jnp.full((1, HW), jnp.inf, dtype=jnp.float32)
    best_i = jnp.zeros((1, HW), dtype=jnp.int32)
    for t in range(T):
        a = z - g_ref[t]
        sq = a * a
        d = (jnp.sum(sq[:, :128], axis=1)
             + jnp.sum(sq[:, 128:], axis=1)).reshape(1, HW)
        i = ids_vec_ref[0, t : t + 1, :]
        take = (d < best_d) | ((d == best_d) & (i < best_i))
        best_d = jnp.where(take, d, best_d)
        best_i = jnp.where(take, i, best_i)
    out_ref[0] = best_i


@jax.jit
def kernel(z_e_x, emb):
    B = z_e_x.shape[0]
    H, W = z_e_x.shape[2], z_e_x.shape[3]
    zb = z_e_x.reshape(B, D, H * W)

    ids = pl.pallas_call(
        _screen_kernel,
        grid=(B,),
        in_specs=[
            pl.BlockSpec((1, D, H * W), lambda b: (b, 0, 0)),
            pl.BlockSpec((K, D), lambda b: (0, 0)),
        ],
        out_specs=pl.BlockSpec((1, T, H * W), lambda b: (b, 0, 0)),
        out_shape=jax.ShapeDtypeStruct((B, T, H * W), jnp.int32),
        scratch_shapes=[
            pltpu.VMEM((K, H * W), jnp.float32),
        ],
        compiler_params=pltpu.CompilerParams(
            dimension_semantics=("parallel",),
        ),
    )(zb, emb)

    zt = zb.transpose(0, 2, 1)

    out = pl.pallas_call(
        _refine_kernel,
        grid=(B,),
        in_specs=[
            pl.BlockSpec((1, T, H * W), lambda b: (b, 0, 0),
                         memory_space=pltpu.SMEM),
            pl.BlockSpec((1, H * W, D), lambda b: (b, 0, 0)),
            pl.BlockSpec((K, D), lambda b: (0, 0)),
            pl.BlockSpec((1, T, H * W), lambda b: (b, 0, 0)),
        ],
        out_specs=pl.BlockSpec((1, 1, H * W), lambda b: (b, 0, 0)),
        out_shape=jax.ShapeDtypeStruct((B, 1, H * W), jnp.int32),
        scratch_shapes=[pltpu.VMEM((T, H * W, D), jnp.float32)],
        compiler_params=pltpu.CompilerParams(
            dimension_semantics=("parallel",),
        ),
    )(ids, zt, emb, ids)
    return out.reshape(B, H, W)


# speedup vs baseline: 15.2490x; 1.2903x over previous
import functools

import jax
from jax import lax
import jax.numpy as jnp
from jax.experimental import pallas as pl
from jax.experimental.pallas import tpu as pltpu
from jax.experimental.pallas import tpu_sc as plsc

K = 512
D = 256
HW = 256
T = 4

NC = 2
NS = 16
NW = NC * NS
CHUNK = 128


def _screen_kernel(zb_ref, emb_ref, ids_ref, s_ref):
    zb = zb_ref[0]
    emb = emb_ref[...]

    esq = emb * emb
    e2 = (jnp.sum(esq[:, :128], axis=1) + jnp.sum(esq[:, 128:], axis=1))

    s = jax.lax.dot_general(
        emb, zb, dimension_numbers=(((1,), (0,)), ((), ())),
        precision=jax.lax.Precision.HIGHEST,
        preferred_element_type=jnp.float32,
    )
    s_ref[...] = e2.reshape(K, 1) - 2.0 * s

    def body(t, carry):
        s = s_ref[...]
        kiota = jax.lax.broadcasted_iota(jnp.int32, (K, HW), 0).astype(
            jnp.float32)
        mv = jnp.min(s, axis=0, keepdims=True)
        idx = jnp.min(jnp.where(s == mv, kiota, jnp.float32(K)),
                      axis=0, keepdims=True)
        ids_ref[0, pl.ds(t, 1), :] = idx.astype(jnp.int32)
        s_ref[...] = jnp.where(kiota == idx, jnp.float32(1e30), s)
        return carry

    jax.lax.fori_loop(0, T, body, 0)


def _sc_gather_kernel(emb_hbm, idx_hbm, out_hbm, idx_v, rows_v, sem):
    wid = lax.axis_index("s") * NC + lax.axis_index("c")
    n = idx_hbm.shape[0] // NW
    base = wid * n
    for c in range(n // CHUNK):
        off = base + c * CHUNK
        pltpu.sync_copy(idx_hbm.at[pl.ds(off, CHUNK)], idx_v)
        pltpu.async_copy(emb_hbm.at[idx_v], rows_v, sem).wait()
        pltpu.sync_copy(rows_v, out_hbm.at[pl.ds(off, CHUNK)])


def _sc_gather(emb, flat_ids):
    n_ids = flat_ids.shape[0]
    k = functools.partial(
        pl.kernel,
        mesh=plsc.VectorSubcoreMesh(core_axis_name="c", subcore_axis_name="s"),
        out_type=jax.ShapeDtypeStruct((n_ids, D), jnp.float32),
        scratch_types=[
            pltpu.VMEM((CHUNK,), jnp.int32),
            pltpu.VMEM((CHUNK, D), jnp.float32),
            pltpu.SemaphoreType.DMA,
        ],
    )(_sc_gather_kernel)
    return k(emb, flat_ids)


def _refine_kernel(z_ref, g_ref, ids_vec_ref, out_ref):
    z = z_ref[0]

    best_d = jnp.full((1, HW), jnp.inf, dtype=jnp.float32)
    best_i = jnp.zeros((1, HW), dtype=jnp.int32)
    for t in range(T):
        a = z - g_ref[0, t]
        sq = a * a
        d = (jnp.sum(sq[:, :128], axis=1)
             + jnp.sum(sq[:, 128:], axis=1)).reshape(1, HW)
        i = ids_vec_ref[0, t : t + 1, :]
        take = (d < best_d) | ((d == best_d) & (i < best_i))
        best_d = jnp.where(take, d, best_d)
        best_i = jnp.where(take, i, best_i)
    out_ref[0] = best_i


@jax.jit
def kernel(z_e_x, emb):
    B = z_e_x.shape[0]
    H, W = z_e_x.shape[2], z_e_x.shape[3]
    zb = z_e_x.reshape(B, D, H * W)

    ids = pl.pallas_call(
        _screen_kernel,
        grid=(B,),
        in_specs=[
            pl.BlockSpec((1, D, H * W), lambda b: (b, 0, 0)),
            pl.BlockSpec((K, D), lambda b: (0, 0)),
        ],
        out_specs=pl.BlockSpec((1, T, H * W), lambda b: (b, 0, 0)),
        out_shape=jax.ShapeDtypeStruct((B, T, H * W), jnp.int32),
        scratch_shapes=[
            pltpu.VMEM((K, H * W), jnp.float32),
        ],
        compiler_params=pltpu.CompilerParams(
            dimension_semantics=("parallel",),
        ),
    )(zb, emb)

    g = _sc_gather(emb, ids.reshape(-1))
    gr = g.reshape(B, T, H * W, D)

    zt = zb.transpose(0, 2, 1)

    out = pl.pallas_call(
        _refine_kernel,
        grid=(B,),
        in_specs=[
            pl.BlockSpec((1, H * W, D), lambda b: (b, 0, 0)),
            pl.BlockSpec((1, T, H * W, D), lambda b: (b, 0, 0, 0)),
            pl.BlockSpec((1, T, H * W), lambda b: (b, 0, 0)),
        ],
        out_specs=pl.BlockSpec((1, 1, H * W), lambda b: (b, 0, 0)),
        out_shape=jax.ShapeDtypeStruct((B, 1, H * W), jnp.int32),
        compiler_params=pltpu.CompilerParams(
            dimension_semantics=("parallel",),
        ),
    )(zt, gr, ids)
    return out.reshape(B, H, W)


# speedup vs baseline: 15.9401x; 1.0453x over previous
import functools

import jax
from jax import lax
import jax.numpy as jnp
from jax.experimental import pallas as pl
from jax.experimental.pallas import tpu as pltpu
from jax.experimental.pallas import tpu_sc as plsc

K = 512
D = 256
HW = 256
T = 4

NC = 2
NS = 16
NW = NC * NS
CHUNK = 128


def _screen_kernel(zb_ref, emb_ref, ids_ref, s_ref):
    zb = zb_ref[0]
    emb = emb_ref[...]

    esq = emb * emb
    e2 = (jnp.sum(esq[:, :128], axis=1) + jnp.sum(esq[:, 128:], axis=1))

    zh = zb.astype(jnp.bfloat16)
    zl = (zb - zh.astype(jnp.float32)).astype(jnp.bfloat16)
    eh = emb.astype(jnp.bfloat16)
    el = (emb - eh.astype(jnp.float32)).astype(jnp.bfloat16)

    def _dot(a, b):
        return jax.lax.dot_general(
            a, b, dimension_numbers=(((1,), (0,)), ((), ())),
            preferred_element_type=jnp.float32)

    s = _dot(eh, zh) + (_dot(eh, zl) + _dot(el, zh))
    s_ref[...] = e2.reshape(K, 1) - 2.0 * s

    def body(t, carry):
        s = s_ref[...]
        kiota = jax.lax.broadcasted_iota(jnp.int32, (K, HW), 0).astype(
            jnp.float32)
        mv = jnp.min(s, axis=0, keepdims=True)
        idx = jnp.min(jnp.where(s == mv, kiota, jnp.float32(K)),
                      axis=0, keepdims=True)
        ids_ref[0, pl.ds(t, 1), :] = idx.astype(jnp.int32)
        s_ref[...] = jnp.where(kiota == idx, jnp.float32(1e30), s)
        return carry

    jax.lax.fori_loop(0, T, body, 0)


def _sc_gather_kernel(emb_hbm, idx_hbm, out_hbm, idx_v, rows_v, sem):
    wid = lax.axis_index("s") * NC + lax.axis_index("c")
    n = idx_hbm.shape[0] // NW
    base = wid * n
    for c in range(n // CHUNK):
        off = base + c * CHUNK
        pltpu.sync_copy(idx_hbm.at[pl.ds(off, CHUNK)], idx_v)
        pltpu.async_copy(emb_hbm.at[idx_v], rows_v, sem).wait()
        pltpu.sync_copy(rows_v, out_hbm.at[pl.ds(off, CHUNK)])


def _sc_gather(emb, flat_ids):
    n_ids = flat_ids.shape[0]
    k = functools.partial(
        pl.kernel,
        mesh=plsc.VectorSubcoreMesh(core_axis_name="c", subcore_axis_name="s"),
        out_type=jax.ShapeDtypeStruct((n_ids, D), jnp.float32),
        scratch_types=[
            pltpu.VMEM((CHUNK,), jnp.int32),
            pltpu.VMEM((CHUNK, D), jnp.float32),
            pltpu.SemaphoreType.DMA,
        ],
    )(_sc_gather_kernel)
    return k(emb, flat_ids)


def _refine_kernel(z_ref, g_ref, ids_vec_ref, out_ref):
    z = z_ref[0]

    best_d = jnp.full((1, HW), jnp.inf, dtype=jnp.float32)
    best_i = jnp.zeros((1, HW), dtype=jnp.int32)
    for t in range(T):
        a = z - g_ref[0, t]
        sq = a * a
        d = (jnp.sum(sq[:, :128], axis=1)
             + jnp.sum(sq[:, 128:], axis=1)).reshape(1, HW)
        i = ids_vec_ref[0, t : t + 1, :]
        take = (d < best_d) | ((d == best_d) & (i < best_i))
        best_d = jnp.where(take, d, best_d)
        best_i = jnp.where(take, i, best_i)
    out_ref[0] = best_i


@jax.jit
def kernel(z_e_x, emb):
    B = z_e_x.shape[0]
    H, W = z_e_x.shape[2], z_e_x.shape[3]
    zb = z_e_x.reshape(B, D, H * W)

    ids = pl.pallas_call(
        _screen_kernel,
        grid=(B,),
        in_specs=[
            pl.BlockSpec((1, D, H * W), lambda b: (b, 0, 0)),
            pl.BlockSpec((K, D), lambda b: (0, 0)),
        ],
        out_specs=pl.BlockSpec((1, T, H * W), lambda b: (b, 0, 0)),
        out_shape=jax.ShapeDtypeStruct((B, T, H * W), jnp.int32),
        scratch_shapes=[
            pltpu.VMEM((K, H * W), jnp.float32),
        ],
        compiler_params=pltpu.CompilerParams(
            dimension_semantics=("parallel",),
        ),
    )(zb, emb)

    g = _sc_gather(emb, ids.reshape(-1))
    gr = g.reshape(B, T, H * W, D)

    zt = zb.transpose(0, 2, 1)

    out = pl.pallas_call(
        _refine_kernel,
        grid=(B,),
        in_specs=[
            pl.BlockSpec((1, H * W, D), lambda b: (b, 0, 0)),
            pl.BlockSpec((1, T, H * W, D), lambda b: (b, 0, 0, 0)),
            pl.BlockSpec((1, T, H * W), lambda b: (b, 0, 0)),
        ],
        out_specs=pl.BlockSpec((1, 1, H * W), lambda b: (b, 0, 0)),
        out_shape=jax.ShapeDtypeStruct((B, 1, H * W), jnp.int32),
        compiler_params=pltpu.CompilerParams(
            dimension_semantics=("parallel",),
        ),
    )(zt, gr, ids)
    return out.reshape(B, H, W)


# speedup vs baseline: 18.0434x; 1.1319x over previous
import functools

import jax
from jax import lax
import jax.numpy as jnp
from jax.experimental import pallas as pl
from jax.experimental.pallas import tpu as pltpu
from jax.experimental.pallas import tpu_sc as plsc

K = 512
D = 256
HW = 256
T = 4

NC = 2
NS = 16
NW = NC * NS
CHUNK = 128


def _screen_kernel(zb_ref, emb_ref, ids_ref, s_ref):
    zb = zb_ref[0]
    emb = emb_ref[...]

    esq = emb * emb
    e2 = (jnp.sum(esq[:, :128], axis=1) + jnp.sum(esq[:, 128:], axis=1))

    zh = zb.astype(jnp.bfloat16)
    zl = (zb - zh.astype(jnp.float32)).astype(jnp.bfloat16)
    eh = emb.astype(jnp.bfloat16)
    el = (emb - eh.astype(jnp.float32)).astype(jnp.bfloat16)

    def _dot(a, b):
        return jax.lax.dot_general(
            a, b, dimension_numbers=(((1,), (0,)), ((), ())),
            preferred_element_type=jnp.float32)

    s = _dot(eh, zh) + (_dot(eh, zl) + _dot(el, zh))
    s_ref[...] = e2.reshape(K, 1) - 2.0 * s

    def body(t, carry):
        s = s_ref[...]
        kiota = jax.lax.broadcasted_iota(jnp.int32, (K, HW), 0).astype(
            jnp.float32)
        mv = jnp.min(s, axis=0, keepdims=True)
        idx = jnp.min(jnp.where(s == mv, kiota, jnp.float32(K)),
                      axis=0, keepdims=True)
        ids_ref[0, pl.ds(t, 1), :] = idx.astype(jnp.int32)
        s_ref[...] = jnp.where(kiota == idx, jnp.float32(1e30), s)
        return carry

    jax.lax.fori_loop(0, T, body, 0)


def _sc_gather_kernel(emb_hbm, idx_hbm, out_hbm, idx_v, rows_v, sem):
    wid = lax.axis_index("s") * NC + lax.axis_index("c")
    n = idx_hbm.shape[0] // NW
    base = wid * n
    for c in range(n // CHUNK):
        off = base + c * CHUNK
        pltpu.sync_copy(idx_hbm.at[pl.ds(off, CHUNK)], idx_v)
        pltpu.async_copy(emb_hbm.at[idx_v], rows_v, sem).wait()
        pltpu.sync_copy(rows_v, out_hbm.at[pl.ds(off, CHUNK)])


def _sc_gather(emb, flat_ids):
    n_ids = flat_ids.shape[0]
    k = functools.partial(
        pl.kernel,
        mesh=plsc.VectorSubcoreMesh(core_axis_name="c", subcore_axis_name="s"),
        out_type=jax.ShapeDtypeStruct((n_ids, D), jnp.float32),
        scratch_types=[
            pltpu.VMEM((CHUNK,), jnp.int32),
            pltpu.VMEM((CHUNK, D), jnp.float32),
            pltpu.SemaphoreType.DMA,
        ],
    )(_sc_gather_kernel)
    return k(emb, flat_ids)


def _refine_kernel(z_ref, g_ref, ids_vec_ref, out_ref):
    z = z_ref[0]

    best_d = jnp.full((1, HW), jnp.inf, dtype=jnp.float32)
    best_i = jnp.zeros((1, HW), dtype=jnp.int32)
    for t in range(T):
        a = z - g_ref[0, t]
        sq = a * a
        d = (jnp.sum(sq[:, :128], axis=1)
             + jnp.sum(sq[:, 128:], axis=1)).reshape(1, HW)
        i = ids_vec_ref[0, t : t + 1, :]
        take = (d < best_d) | ((d == best_d) & (i < best_i))
        best_d = jnp.where(take, d, best_d)
        best_i = jnp.where(take, i, best_i)
    out_ref[0] = best_i


def _screen(zb, emb):
    b = zb.shape[0]
    hw = zb.shape[2]
    return pl.pallas_call(
        _screen_kernel,
        grid=(b,),
        in_specs=[
            pl.BlockSpec((1, D, hw), lambda i: (i, 0, 0)),
            pl.BlockSpec((K, D), lambda i: (0, 0)),
        ],
        out_specs=pl.BlockSpec((1, T, hw), lambda i: (i, 0, 0)),
        out_shape=jax.ShapeDtypeStruct((b, T, hw), jnp.int32),
        scratch_shapes=[
            pltpu.VMEM((K, hw), jnp.float32),
        ],
        compiler_params=pltpu.CompilerParams(
            dimension_semantics=("parallel",),
        ),
    )(zb, emb)


def _refine(zt, gr, ids):
    b = zt.shape[0]
    hw = zt.shape[1]
    return pl.pallas_call(
        _refine_kernel,
        grid=(b,),
        in_specs=[
            pl.BlockSpec((1, hw, D), lambda i: (i, 0, 0)),
            pl.BlockSpec((1, T, hw, D), lambda i: (i, 0, 0, 0)),
            pl.BlockSpec((1, T, hw), lambda i: (i, 0, 0)),
        ],
        out_specs=pl.BlockSpec((1, 1, hw), lambda i: (i, 0, 0)),
        out_shape=jax.ShapeDtypeStruct((b, 1, hw), jnp.int32),
        compiler_params=pltpu.CompilerParams(
            dimension_semantics=("parallel",),
        ),
    )(zt, gr, ids)


@jax.jit
def kernel(z_e_x, emb):
    B = z_e_x.shape[0]
    H, W = z_e_x.shape[2], z_e_x.shape[3]
    zb = z_e_x.reshape(B, D, H * W)

    halves = [zb[: B // 2], zb[B // 2 :]] if B % 2 == 0 else [zb]
    ids_h = [_screen(h, emb) for h in halves]
    g_h = [_sc_gather(emb, ids.reshape(-1)) for ids in ids_h]
    outs = []
    for h, ids, g in zip(halves, ids_h, g_h):
        b = h.shape[0]
        gr = g.reshape(b, T, H * W, D)
        zt = h.transpose(0, 2, 1)
        outs.append(_refine(zt, gr, ids))
    out = jnp.concatenate(outs, axis=0) if len(outs) > 1 else outs[0]
    return out.reshape(B, H, W)
